# Initial kernel scaffold; baseline (speedup 1.0000x reference)
import functools
from math import factorial

import jax
import jax.numpy as jnp
import numpy as np
from jax.experimental import pallas as pl
from jax.experimental.pallas import tpu as pltpu

_MAX_ATOM = 100000
_NUM_BASIS = 8
_MAX_RADIUS = 6.0
_PATHS = [(0, 0, 0), (0, 1, 1), (0, 2, 2), (1, 0, 1), (1, 1, 0), (1, 1, 2),
          (1, 2, 1), (2, 0, 2), (2, 1, 1), (2, 2, 0), (2, 2, 2), (3, 1, 2),
          (3, 2, 1)]
_OFF1 = {0: 0, 1: 1, 2: 4, 3: 9}
_OFF2 = {0: 0, 1: 1, 2: 4}
_FAN_IN = {0: 3.0, 1: 5.0, 2: 5.0}
_SILU_CST = 1.679177


def _su2_cg_coef(j1, m1, j2, m2, j3, m3):
    if m3 != m1 + m2:
        return 0.0
    f = lambda n: float(factorial(round(n)))
    vmin = int(max(-j1 + j2 + m3, -j1 + m1, 0))
    vmax = int(min(j2 + j3 + m1, j3 - j1 + j2, j3 + m3))
    C = ((2 * j3 + 1) * f(j3 + j1 - j2) * f(j3 - j1 + j2) * f(j1 + j2 - j3)
         / f(j1 + j2 + j3 + 1)) ** 0.5
    C = C * (f(j3 + m3) * f(j3 - m3)
             / (f(j1 + m1) * f(j1 - m1) * f(j2 + m2) * f(j2 - m2))) ** 0.5
    S = 0.0
    for v in range(vmin, vmax + 1):
        S = S + (-1.0) ** (v + j2 + m2) / f(v) * f(j2 + j3 + m1 - v) \
            * f(j1 - m1 + v) / f(j3 - j1 + j2 - v) / f(j3 + m3 - v) \
            / f(v + j1 - j2 - m3)
    return C * S


def _su2_cg(j1, j2, j3):
    mat = np.zeros((2 * j1 + 1, 2 * j2 + 1, 2 * j3 + 1))
    for m1 in range(-j1, j1 + 1):
        for m2 in range(-j2, j2 + 1):
            m3 = m1 + m2
            if abs(m3) <= j3:
                mat[j1 + m1, j2 + m2, j3 + m3] = _su2_cg_coef(j1, m1, j2, m2, j3, m3)
    return mat


def _q_mat(l):
    q = np.zeros((2 * l + 1, 2 * l + 1), dtype=np.complex128)
    for m in range(-l, 0):
        q[l + m, l + abs(m)] = 1.0 / 2 ** 0.5
        q[l + m, l - abs(m)] = -1j / 2 ** 0.5
    q[l, l] = 1.0
    for m in range(1, l + 1):
        q[l + m, l + abs(m)] = (-1) ** m / 2 ** 0.5
        q[l + m, l - abs(m)] = 1j * (-1) ** m / 2 ** 0.5
    return (-1j) ** l * q


def _wigner_3j(l1, l2, l3):
    C = _su2_cg(l1, l2, l3).astype(np.complex128)
    C = np.einsum('ij,kl,mn,ikn->jlm', _q_mat(l1), _q_mat(l2),
                  np.conj(_q_mat(l3).T), C)
    C = np.real(C)
    return (C / np.linalg.norm(C)).astype(np.float32)


def _build_tables():
    w3j = {p: _wigner_3j(*p) for p in sorted(set(_PATHS))}
    rows = []
    for p, (l1, l2, l3) in enumerate(_PATHS):
        for jl in range(2 * l2 + 1):
            rows.append((p, l1, l2, l3, jl))
    R = len(rows)
    M = np.zeros((16, R, 9), dtype=np.float32)
    A = np.zeros((R, 13), dtype=np.float32)
    B = np.zeros((R, 9), dtype=np.float32)
    for r, (p, l1, l2, l3, jl) in enumerate(rows):
        alpha = ((2 * l3 + 1) / _FAN_IN[l3]) ** 0.5
        C = w3j[(l1, l2, l3)]
        for il in range(2 * l1 + 1):
            for kl in range(2 * l3 + 1):
                M[_OFF1[l1] + il, r, _OFF2[l3] + kl] += alpha * C[il, jl, kl]
        A[r, p] = 1.0
        B[r, _OFF2[l2] + jl] = 1.0
    return M, A, B


_M_NP, _A_NP, _B_NP = _build_tables()
_NROWS = _A_NP.shape[0]

_RBF_STEP = _MAX_RADIUS / (_NUM_BASIS + 1)
_RBF_CENTERS = np.arange(1, _NUM_BASIS + 1, dtype=np.float32) * _RBF_STEP

_BLOCK = 2048


def _conv_body(f_ref, posT_ref, w1t_ref, w2t_ref, out_ref, t2_ref):
    f32 = jnp.float32

    @pl.when(pl.program_id(0) == 0)
    def _():
        t2 = jnp.zeros((_NROWS, 9), dtype=f32)
        for i in range(16):
            t2 = t2 + f_ref[0:1, i:i + 1] * jnp.asarray(_M_NP[i])
        t2_ref[...] = t2

    pT = posT_ref[...]
    x = pT[0:1, :]
    y = pT[1:2, :]
    z = pT[2:3, :]
    r2 = x * x + y * y + z * z
    r = jnp.sqrt(r2)
    inv_r = 1.0 / r
    ux = x * inv_r
    uy = y * inv_r
    uz = z * inv_r

    s3 = 3.0 ** 0.5
    s15 = 15.0 ** 0.5
    sh = jnp.concatenate([
        jnp.ones_like(ux),
        s3 * uy, s3 * uz, s3 * ux,
        s15 * ux * uy,
        s15 * uy * uz,
        (5.0 ** 0.5 / 2.0) * (3.0 * uz * uz - 1.0),
        s15 * ux * uz,
        (s15 / 2.0) * (ux * ux - uy * uy),
    ], axis=0)

    centers = jnp.asarray(_RBF_CENTERS.reshape(_NUM_BASIS, 1))
    diff = (r - centers) * (1.0 / _RBF_STEP)
    emb = jnp.exp(-diff * diff) * (_NUM_BASIS ** 0.5 / 1.12)

    a = jnp.dot(w1t_ref[...], emb, preferred_element_type=f32) \
        * (1.0 / 8.0 ** 0.5)
    h = _SILU_CST * a * jax.nn.sigmoid(a)
    w = jnp.dot(w2t_ref[...], h, preferred_element_type=f32) \
        * (1.0 / 32.0 ** 0.5)

    V = jnp.dot(jnp.asarray(_A_NP), w, preferred_element_type=f32) \
        * jnp.dot(jnp.asarray(_B_NP), sh, preferred_element_type=f32)

    out_ref[...] = jax.lax.dot_general(
        V, t2_ref[...], dimension_numbers=(((0,), (0,)), ((), ())),
        preferred_element_type=f32)


@jax.jit
def kernel(f_in, pos, fc_w1, fc_w2):
    n = pos.shape[0]
    grid = pl.cdiv(n, _BLOCK)
    posT = pos.T
    w1t = fc_w1.T
    w2t = fc_w2.T
    out = pl.pallas_call(
        _conv_body,
        grid=(grid,),
        in_specs=[
            pl.BlockSpec((1, 16), lambda i: (0, 0)),
            pl.BlockSpec((3, _BLOCK), lambda i: (0, i)),
            pl.BlockSpec((32, 8), lambda i: (0, 0)),
            pl.BlockSpec((13, 32), lambda i: (0, 0)),
        ],
        out_specs=pl.BlockSpec((_BLOCK, 9), lambda i: (i, 0)),
        out_shape=jax.ShapeDtypeStruct((n, 9), jnp.float32),
        scratch_shapes=[pltpu.VMEM((_NROWS, 9), jnp.float32)],
    )(f_in, posT, w1t, w2t)
    return out


# speedup vs baseline: 44.1966x; 44.1966x over previous
import functools
from math import factorial

import jax
import jax.numpy as jnp
import numpy as np
from jax.experimental import pallas as pl
from jax.experimental.pallas import tpu as pltpu

_MAX_ATOM = 100000
_NUM_BASIS = 8
_MAX_RADIUS = 6.0
_PATHS = [(0, 0, 0), (0, 1, 1), (0, 2, 2), (1, 0, 1), (1, 1, 0), (1, 1, 2),
          (1, 2, 1), (2, 0, 2), (2, 1, 1), (2, 2, 0), (2, 2, 2), (3, 1, 2),
          (3, 2, 1)]
_OFF1 = {0: 0, 1: 1, 2: 4, 3: 9}
_OFF2 = {0: 0, 1: 1, 2: 4}
_FAN_IN = {0: 3.0, 1: 5.0, 2: 5.0}
_SILU_CST = 1.679177


def _su2_cg_coef(j1, m1, j2, m2, j3, m3):
    if m3 != m1 + m2:
        return 0.0
    f = lambda n: float(factorial(round(n)))
    vmin = int(max(-j1 + j2 + m3, -j1 + m1, 0))
    vmax = int(min(j2 + j3 + m1, j3 - j1 + j2, j3 + m3))
    C = ((2 * j3 + 1) * f(j3 + j1 - j2) * f(j3 - j1 + j2) * f(j1 + j2 - j3)
         / f(j1 + j2 + j3 + 1)) ** 0.5
    C = C * (f(j3 + m3) * f(j3 - m3)
             / (f(j1 + m1) * f(j1 - m1) * f(j2 + m2) * f(j2 - m2))) ** 0.5
    S = 0.0
    for v in range(vmin, vmax + 1):
        S = S + (-1.0) ** (v + j2 + m2) / f(v) * f(j2 + j3 + m1 - v) \
            * f(j1 - m1 + v) / f(j3 - j1 + j2 - v) / f(j3 + m3 - v) \
            / f(v + j1 - j2 - m3)
    return C * S


def _su2_cg(j1, j2, j3):
    mat = np.zeros((2 * j1 + 1, 2 * j2 + 1, 2 * j3 + 1))
    for m1 in range(-j1, j1 + 1):
        for m2 in range(-j2, j2 + 1):
            m3 = m1 + m2
            if abs(m3) <= j3:
                mat[j1 + m1, j2 + m2, j3 + m3] = _su2_cg_coef(j1, m1, j2, m2, j3, m3)
    return mat


def _q_mat(l):
    q = np.zeros((2 * l + 1, 2 * l + 1), dtype=np.complex128)
    for m in range(-l, 0):
        q[l + m, l + abs(m)] = 1.0 / 2 ** 0.5
        q[l + m, l - abs(m)] = -1j / 2 ** 0.5
    q[l, l] = 1.0
    for m in range(1, l + 1):
        q[l + m, l + abs(m)] = (-1) ** m / 2 ** 0.5
        q[l + m, l - abs(m)] = 1j * (-1) ** m / 2 ** 0.5
    return (-1j) ** l * q


def _wigner_3j(l1, l2, l3):
    C = _su2_cg(l1, l2, l3).astype(np.complex128)
    C = np.einsum('ij,kl,mn,ikn->jlm', _q_mat(l1), _q_mat(l2),
                  np.conj(_q_mat(l3).T), C)
    C = np.real(C)
    return (C / np.linalg.norm(C)).astype(np.float32)


def _build_tables():
    w3j = {p: _wigner_3j(*p) for p in sorted(set(_PATHS))}
    rows = []
    for p, (l1, l2, l3) in enumerate(_PATHS):
        for jl in range(2 * l2 + 1):
            rows.append((p, l1, l2, l3, jl))
    R = len(rows)
    M = np.zeros((16, R, 9), dtype=np.float32)
    A = np.zeros((R, 13), dtype=np.float32)
    B = np.zeros((R, 9), dtype=np.float32)
    for r, (p, l1, l2, l3, jl) in enumerate(rows):
        alpha = ((2 * l3 + 1) / _FAN_IN[l3]) ** 0.5
        C = w3j[(l1, l2, l3)]
        for il in range(2 * l1 + 1):
            for kl in range(2 * l3 + 1):
                M[_OFF1[l1] + il, r, _OFF2[l3] + kl] += alpha * C[il, jl, kl]
        A[r, p] = 1.0
        B[r, _OFF2[l2] + jl] = 1.0
    return M, A, B


_M_NP, _A_NP, _B_NP = _build_tables()
_NROWS = _A_NP.shape[0]

_RBF_STEP = _MAX_RADIUS / (_NUM_BASIS + 1)
_RBF_CENTERS = np.arange(1, _NUM_BASIS + 1, dtype=np.float32) * _RBF_STEP

_BLOCK = 2048


def _conv_body(f_ref, posT_ref, w1t_ref, w2t_ref, m_ref, a_ref, b_ref,
               out_ref, t2_ref):
    f32 = jnp.float32

    @pl.when(pl.program_id(0) == 0)
    def _():
        t2 = jnp.zeros((_NROWS, 9), dtype=f32)
        for i in range(16):
            t2 = t2 + f_ref[0:1, i:i + 1] * m_ref[i]
        t2_ref[...] = t2

    pT = posT_ref[...]
    x = pT[0:1, :]
    y = pT[1:2, :]
    z = pT[2:3, :]
    r2 = x * x + y * y + z * z
    r = jnp.sqrt(r2)
    inv_r = 1.0 / r
    ux = x * inv_r
    uy = y * inv_r
    uz = z * inv_r

    s3 = 3.0 ** 0.5
    s15 = 15.0 ** 0.5
    sh = jnp.concatenate([
        jnp.ones_like(ux),
        s3 * uy, s3 * uz, s3 * ux,
        s15 * ux * uy,
        s15 * uy * uz,
        (5.0 ** 0.5 / 2.0) * (3.0 * uz * uz - 1.0),
        s15 * ux * uz,
        (s15 / 2.0) * (ux * ux - uy * uy),
    ], axis=0)

    centers = (jax.lax.broadcasted_iota(jnp.int32, (_NUM_BASIS, 1), 0)
               .astype(f32) + 1.0) * _RBF_STEP
    diff = (r - centers) * (1.0 / _RBF_STEP)
    emb = jnp.exp(-diff * diff) * (_NUM_BASIS ** 0.5 / 1.12)

    a = jnp.dot(w1t_ref[...], emb, preferred_element_type=f32) \
        * (1.0 / 8.0 ** 0.5)
    h = _SILU_CST * a * jax.nn.sigmoid(a)
    w = jnp.dot(w2t_ref[...], h, preferred_element_type=f32) \
        * (1.0 / 32.0 ** 0.5)

    V = jnp.dot(a_ref[...], w, preferred_element_type=f32) \
        * jnp.dot(b_ref[...], sh, preferred_element_type=f32)

    out_ref[...] = jax.lax.dot_general(
        V, t2_ref[...], dimension_numbers=(((0,), (0,)), ((), ())),
        preferred_element_type=f32)


@jax.jit
def kernel(f_in, pos, fc_w1, fc_w2):
    n = pos.shape[0]
    grid = pl.cdiv(n, _BLOCK)
    posT = pos.T
    w1t = fc_w1.T
    w2t = fc_w2.T
    out = pl.pallas_call(
        _conv_body,
        grid=(grid,),
        in_specs=[
            pl.BlockSpec((8, 16), lambda i: (0, 0)),
            pl.BlockSpec((3, _BLOCK), lambda i: (0, i)),
            pl.BlockSpec((32, 8), lambda i: (0, 0)),
            pl.BlockSpec((13, 32), lambda i: (0, 0)),
            pl.BlockSpec((16, _NROWS, 9), lambda i: (0, 0, 0)),
            pl.BlockSpec((_NROWS, 13), lambda i: (0, 0)),
            pl.BlockSpec((_NROWS, 9), lambda i: (0, 0)),
        ],
        out_specs=pl.BlockSpec((_BLOCK, 9), lambda i: (i, 0)),
        out_shape=jax.ShapeDtypeStruct((n, 9), jnp.float32),
        scratch_shapes=[pltpu.VMEM((_NROWS, 9), jnp.float32)],
    )(f_in, posT, w1t, w2t, jnp.asarray(_M_NP), jnp.asarray(_A_NP),
      jnp.asarray(_B_NP))
    return out
